# Initial kernel scaffold; baseline (speedup 1.0000x reference)
import jax
import jax.numpy as jnp
from jax.experimental import pallas as pl


def kernel(x, W_feat, W_off, Wq, Wk, Wv, Wo):
    raise NotImplementedError("write your pallas kernel here")


# speedup vs baseline: 2.2206x; 2.2206x over previous
import jax
import jax.numpy as jnp
from jax import lax
from jax.experimental import pallas as pl
from jax.experimental.pallas import tpu as pltpu
from jax.experimental.pallas import tpu_sc as plsc

R = 16
M = R ** 3
H = 8
B_FIXED = 4

NC = 2
NS = 16
LANES = 16
MS = M // NS

_SC_MESH = dict(core_axis_name="c", subcore_axis_name="s",
                num_cores=NC, num_subcores=NS)
_SC_PARAMS = dict(needs_layout_passes=False)


def _voxelize_body(xt_ref, seg_ref, norm_ref):
    xs = xt_ref[0]
    mn = jnp.min(xs, axis=1, keepdims=True)
    mx = jnp.max(xs, axis=1, keepdims=True)
    norm = (xs - mn) / (mx - mn + 1e-9)
    v = jnp.clip(jnp.floor(norm * R).astype(jnp.int32), 0, R - 1)
    seg_ref[0] = v[0:1] * (R * R) + v[1:2] * R + v[2:3]
    norm_ref[0] = norm


def _voxelize(xt):
    B, _, N = xt.shape
    return pl.pallas_call(
        _voxelize_body,
        grid=(B,),
        in_specs=[pl.BlockSpec((1, 3, N), lambda b: (b, 0, 0))],
        out_specs=[
            pl.BlockSpec((1, 1, N), lambda b: (b, 0, 0)),
            pl.BlockSpec((1, 3, N), lambda b: (b, 0, 0)),
        ],
        out_shape=[
            jax.ShapeDtypeStruct((B, 1, N), jnp.int32),
            jax.ShapeDtypeStruct((B, 3, N), jnp.float32),
        ],
    )(xt)


def _stats_body(seg1d, px_h, py_h, pz_h, zeros_h,
                A_out, cnt_out,
                segl_v, px_v, py_v, pz_v, st_p, red_v, tmp_v, fin_t, A_v,
                stage_sh, fin_sh):
    c = lax.axis_index("c")
    s = lax.axis_index("s")
    BN = seg1d.shape[0]
    N = BN // B_FIXED
    npercore = B_FIXED // NC
    npert = N // NS

    pltpu.sync_copy(zeros_h.at[pl.ds(0, npert * LANES)], A_v)

    for r in range(npercore):
        b = c * npercore + r
        base = s * npert
        boff = b * N + base
        pltpu.sync_copy(zeros_h.at[pl.ds(0, 4 * M + LANES)], st_p)
        pltpu.sync_copy(seg1d.at[pl.ds(boff, npert)],
                        segl_v.at[pl.ds(0, npert)])
        pltpu.sync_copy(px_h.at[pl.ds(boff, npert)],
                        px_v.at[pl.ds(0, npert)])
        pltpu.sync_copy(py_h.at[pl.ds(boff, npert)],
                        py_v.at[pl.ds(0, npert)])
        pltpu.sync_copy(pz_h.at[pl.ds(boff, npert)],
                        pz_v.at[pl.ds(0, npert)])

        lane0 = (lax.iota(jnp.int32, LANES) == 0).astype(jnp.float32)

        def pbody(i, _):
            sg = segl_v[pl.ds(i, LANES)][0]
            pxi = px_v[pl.ds(i, LANES)][0]
            pyi = py_v[pl.ds(i, LANES)][0]
            pzi = pz_v[pl.ds(i, LANES)][0]
            sl0 = pl.ds(sg, LANES)
            st_p[sl0] = st_p[sl0] + lane0
            sl1 = pl.ds(M + sg, LANES)
            st_p[sl1] = st_p[sl1] + lane0 * pxi
            sl2 = pl.ds(2 * M + sg, LANES)
            st_p[sl2] = st_p[sl2] + lane0 * pyi
            sl3 = pl.ds(3 * M + sg, LANES)
            st_p[sl3] = st_p[sl3] + lane0 * pzi
            return 0

        lax.fori_loop(0, npert, pbody, 0)
        pltpu.sync_copy(st_p.at[pl.ds(0, 4 * M)],
                        stage_sh.at[pl.ds(s * 4 * M, 4 * M)])
        plsc.subcore_barrier()

        pltpu.sync_copy(zeros_h.at[pl.ds(0, 4 * MS)], red_v)
        for a in range(4):
            def rbody(u, _):
                pltpu.sync_copy(
                    stage_sh.at[pl.ds(u * 4 * M + a * M + s * MS, MS)],
                    tmp_v)
                for q in range(MS // LANES):
                    sl = pl.ds(a * MS + q * LANES, LANES)
                    red_v[sl] = red_v[sl] + tmp_v[pl.ds(q * LANES, LANES)]
                return 0

            lax.fori_loop(0, NS, rbody, 0)
        for q in range(MS // LANES):
            d = jnp.maximum(red_v[pl.ds(q * LANES, LANES)], 1.0)
            for a in range(1, 4):
                sl = pl.ds(a * MS + q * LANES, LANES)
                red_v[sl] = red_v[sl] / d
        pltpu.sync_copy(red_v.at[pl.ds(0, MS)],
                        cnt_out.at[pl.ds(b * M + s * MS, MS)])
        for a in range(3):
            pltpu.sync_copy(red_v.at[pl.ds((a + 1) * MS, MS)],
                            fin_sh.at[pl.ds(a * M + s * MS, MS)])
        plsc.subcore_barrier()
        pltpu.sync_copy(fin_sh, fin_t)

        def gat_body(g, _):
            p0 = g * LANES
            idx16 = segl_v[pl.ds(p0, LANES)]
            gx = plsc.load_gather(fin_t, [idx16])
            gy = plsc.load_gather(fin_t, [idx16 + M])
            gz = plsc.load_gather(fin_t, [idx16 + 2 * M])
            px = px_v[pl.ds(p0, LANES)]
            py = py_v[pl.ds(p0, LANES)]
            pz = pz_v[pl.ds(p0, LANES)]
            rows = (p0 + lax.iota(jnp.int32, LANES)) * LANES
            plsc.store_scatter(A_v, [rows + 0], px)
            plsc.store_scatter(A_v, [rows + 1], py)
            plsc.store_scatter(A_v, [rows + 2], pz)
            plsc.store_scatter(A_v, [rows + 3], gx)
            plsc.store_scatter(A_v, [rows + 4], gy)
            plsc.store_scatter(A_v, [rows + 5], gz)
            return 0

        lax.fori_loop(0, npert // LANES, gat_body, 0)
        pltpu.sync_copy(A_v, A_out.at[pl.ds(boff * LANES, npert * LANES)])
        plsc.subcore_barrier()


def _stats(seg1d, px_h, py_h, pz_h, zeros_h):
    BN = seg1d.shape[0]
    npert = BN // B_FIXED // NS
    mesh = plsc.VectorSubcoreMesh(**_SC_MESH)
    f = pl.kernel(
        _stats_body,
        out_type=(
            jax.ShapeDtypeStruct((BN * LANES,), jnp.float32),
            jax.ShapeDtypeStruct((B_FIXED * M,), jnp.float32),
        ),
        mesh=mesh,
        compiler_params=pltpu.CompilerParams(**_SC_PARAMS),
        scratch_types=[
            pltpu.VMEM((npert + LANES,), jnp.int32),
            pltpu.VMEM((npert + LANES,), jnp.float32),
            pltpu.VMEM((npert + LANES,), jnp.float32),
            pltpu.VMEM((npert + LANES,), jnp.float32),
            pltpu.VMEM((4 * M + LANES,), jnp.float32),
            pltpu.VMEM((4 * MS,), jnp.float32),
            pltpu.VMEM((MS,), jnp.float32),
            pltpu.VMEM((3 * M,), jnp.float32),
            pltpu.VMEM((npert * LANES,), jnp.float32),
            pltpu.VMEM_SHARED((NS * 4 * M,), jnp.float32),
            pltpu.VMEM_SHARED((3 * M,), jnp.float32),
        ],
    )
    return f(seg1d, px_h, py_h, pz_h, zeros_h)


def _embed_body(A_ref, W_ref, pe_ref):
    pe_ref[...] = jnp.tanh(
        jnp.dot(A_ref[...], W_ref[...], preferred_element_type=jnp.float32))


def _embed(A2, Wcat):
    BN = A2.shape[0]
    CH = 2048
    D = Wcat.shape[1]
    return pl.pallas_call(
        _embed_body,
        grid=(BN // CH,),
        in_specs=[
            pl.BlockSpec((CH, LANES), lambda n: (n, 0)),
            pl.BlockSpec((LANES, D), lambda n: (0, 0)),
        ],
        out_specs=pl.BlockSpec((CH, D), lambda n: (n, 0)),
        out_shape=jax.ShapeDtypeStruct((BN, D), jnp.float32),
    )(A2, Wcat)


_PIDB = 19


def _scatter_body(seg1d, pe2, zeros_h, vox_out,
                  segc_v, plist, idxb, rows_v, acc):
    c = lax.axis_index("c")
    s = lax.axis_index("s")
    BN, D = pe2.shape
    N = BN // B_FIXED
    npercore = B_FIXED // NC
    SC_CH = 8192
    GCH = 64
    v0 = s * MS
    iota16 = lax.iota(jnp.int32, LANES)

    for r in range(npercore):
        b = c * npercore + r
        pltpu.sync_copy(zeros_h, acc)

        cursor = 0
        for cblk in range(N // SC_CH):
            pltpu.sync_copy(seg1d.at[pl.ds(b * N + cblk * SC_CH, SC_CH)],
                            segc_v)

            def scan_body(g, cur):
                seg16 = segc_v[pl.ds(g * LANES, LANES)]
                pid = (b * N + cblk * SC_CH + g * LANES) + iota16
                loc = seg16 - v0
                mask = (seg16 >= v0) & (seg16 < v0 + MS)
                packed = pid + (loc << _PIDB)
                plsc.store_compressed(plist.at[pl.ds(cur, LANES)],
                                      packed, mask=mask)
                npts = plsc.all_reduce_population_count(mask)
                return cur + jnp.max(npts)

            cursor = lax.fori_loop(0, SC_CH // LANES, scan_body, cursor)
        sent = jnp.full((LANES,), (MS << _PIDB) + b * N, jnp.int32)
        for t in range(GCH // LANES):
            plist[pl.ds(cursor + t * LANES, LANES)] = sent

        def chunk_body(k, _):
            for t in range(GCH // LANES):
                pk = plist[pl.ds(k * GCH + t * LANES, LANES)]
                idxb[pl.ds(t * LANES, LANES)] = pk & ((1 << _PIDB) - 1)
            pltpu.sync_copy(pe2.at[idxb], rows_v)

            def rowadd(i, _):
                pk = plist[pl.ds(k * GCH + i, LANES)][0]
                aoff = (pk >> _PIDB) * D
                for kk in range(D // LANES):
                    plsc.addupdate(
                        acc.at[pl.ds(aoff + kk * LANES, LANES)],
                        rows_v[i, pl.ds(kk * LANES, LANES)])
                return 0

            lax.fori_loop(0, GCH, rowadd, 0)
            return 0

        lax.fori_loop(0, (cursor + GCH - 1) // GCH, chunk_body, 0)
        pltpu.sync_copy(acc.at[pl.ds(0, MS * D)],
                        vox_out.at[pl.ds((b * M + v0) * D, MS * D)])


def _scatter(seg1d, pe2, zeros_h):
    BN, D = pe2.shape
    mesh = plsc.VectorSubcoreMesh(**_SC_MESH)
    f = pl.kernel(
        _scatter_body,
        out_type=jax.ShapeDtypeStruct((B_FIXED * M * D,), jnp.float32),
        mesh=mesh,
        compiler_params=pltpu.CompilerParams(**_SC_PARAMS),
        scratch_types=[
            pltpu.VMEM((8192,), jnp.int32),
            pltpu.VMEM((BN // B_FIXED + 128,), jnp.int32),
            pltpu.VMEM((64,), jnp.int32),
            pltpu.VMEM((64, 256), jnp.float32),
            pltpu.VMEM(((MS + 1) * 256,), jnp.float32),
        ],
    )
    return f(seg1d, pe2, zeros_h)


def _attn_body(vox_ref, cnt_ref, Wq_ref, Wk_ref, Wv_ref, Wo_ref,
               S_ref, ST_ref, out_ref, attn_ref):
    cnt = cnt_ref[0]
    occ = cnt > 0.0
    feat = vox_ref[0] / jnp.maximum(cnt, 1.0)
    q = jnp.dot(feat, Wq_ref[...], preferred_element_type=jnp.float32)
    k = jnp.dot(feat, Wk_ref[...], preferred_element_type=jnp.float32)
    v = jnp.dot(feat, Wv_ref[...], preferred_element_type=jnp.float32)
    dh = Wq_ref.shape[1] // H
    scores = jnp.dot(q * k, S_ref[...],
                     preferred_element_type=jnp.float32) / jnp.sqrt(
                         jnp.float32(dh))
    scores = jnp.where(occ, scores, jnp.float32(-1e9))
    mx = jnp.max(scores, axis=0, keepdims=True)
    e = jnp.exp(scores - mx)
    attn = e / jnp.sum(e, axis=0, keepdims=True)
    attn_ref[0] = attn
    wexp = jnp.dot(attn, ST_ref[...],
                   preferred_element_type=jnp.float32) * v
    of = jnp.dot(wexp, Wo_ref[...], preferred_element_type=jnp.float32)
    of = jnp.where(occ, of, jnp.float32(-1e9))
    out_ref[0] = jnp.max(of, axis=0, keepdims=True)


def _attention(vox, cnt3, Wq, Wk, Wv, Wo, S, ST):
    B, M_, D = vox.shape
    return pl.pallas_call(
        _attn_body,
        grid=(B,),
        in_specs=[
            pl.BlockSpec((1, M_, D), lambda b: (b, 0, 0)),
            pl.BlockSpec((1, M_, 1), lambda b: (b, 0, 0)),
            pl.BlockSpec((D, D), lambda b: (0, 0)),
            pl.BlockSpec((D, D), lambda b: (0, 0)),
            pl.BlockSpec((D, D), lambda b: (0, 0)),
            pl.BlockSpec((D, D), lambda b: (0, 0)),
            pl.BlockSpec((D, H), lambda b: (0, 0)),
            pl.BlockSpec((H, D), lambda b: (0, 0)),
        ],
        out_specs=[
            pl.BlockSpec((1, 1, D), lambda b: (b, 0, 0)),
            pl.BlockSpec((1, M_, H), lambda b: (b, 0, 0)),
        ],
        out_shape=[
            jax.ShapeDtypeStruct((B, 1, D), jnp.float32),
            jax.ShapeDtypeStruct((B, M_, H), jnp.float32),
        ],
    )(vox, cnt3, Wq, Wk, Wv, Wo, S, ST)


def kernel(x, W_feat, W_off, Wq, Wk, Wv, Wo):
    B, N, _ = x.shape
    D = W_feat.shape[1]

    xt = jnp.transpose(x, (0, 2, 1))
    seg, norm = _voxelize(xt)
    seg1d = seg.reshape(B * N)

    nt = jnp.transpose(norm, (1, 0, 2)).reshape(3, B * N)
    zeros_h = jnp.zeros(((MS + 1) * D,), jnp.float32)
    A1, cnt = _stats(seg1d, nt[0], nt[1], nt[2], zeros_h)
    A2 = A1.reshape(B * N, LANES)

    Wcat = jnp.concatenate(
        [W_feat + W_off, -W_off, jnp.zeros((LANES - 6, D), jnp.float32)],
        axis=0)
    pe2 = _embed(A2, Wcat)

    vox1 = _scatter(seg1d, pe2, zeros_h)
    vox = vox1.reshape(B, M, D)

    hid = (jnp.arange(D, dtype=jnp.int32)[:, None] // (D // H)
           == jnp.arange(H, dtype=jnp.int32)[None, :])
    S = hid.astype(jnp.float32)
    ST = S.T
    out, attn = _attention(vox, cnt.reshape(B, M, 1),
                           Wq, Wk, Wv, Wo, S, ST)
    return (out.reshape(B, D), attn)
